# Initial kernel scaffold; baseline (speedup 1.0000x reference)
import jax
import jax.numpy as jnp
from jax.experimental import pallas as pl


def kernel(actions, batch_time_shape, base_token, embedding):
    raise NotImplementedError("write your pallas kernel here")


# speedup vs baseline: 1.3454x; 1.3454x over previous
import functools

import jax
import jax.numpy as jnp
from jax import lax
from jax.experimental import pallas as pl
from jax.experimental.pallas import tpu as pltpu
from jax.experimental.pallas import tpu_sc as plsc

_NC = 2
_NS = 16
_NW = _NC * _NS

_CH = 128
_GR = 8


def _prep_table(base_token, embedding, v_pad, block_rows):
    v, d = embedding.shape

    def body(bt_ref, emb_ref, out_ref):
        i = pl.program_id(0)
        rows = jax.lax.broadcasted_iota(jnp.int32, (block_rows, 1), 0) + i * block_rows
        emb_bf = emb_ref[...].astype(jnp.bfloat16)
        bt_bf = bt_ref[...].astype(jnp.bfloat16)
        masked = jnp.where(rows < v, emb_bf, jnp.zeros((), jnp.bfloat16))
        out_ref[...] = masked + bt_bf

    grid = (v_pad // block_rows,)
    return pl.pallas_call(
        body,
        grid=grid,
        in_specs=[
            pl.BlockSpec((1, d), lambda i: (0, 0)),
            pl.BlockSpec((block_rows, d), lambda i: (i, 0)),
        ],
        out_specs=pl.BlockSpec((block_rows, d), lambda i: (i, 0)),
        out_shape=jax.ShapeDtypeStruct((v_pad, d), jnp.bfloat16),
    )(base_token.reshape(1, d), embedding)


def _remap_idx(actions, v):
    def body(a_ref, o_ref):
        a = a_ref[...]
        o_ref[...] = jnp.where(a >= 0, a, jnp.int32(v))

    return pl.pallas_call(
        body,
        out_shape=jax.ShapeDtypeStruct(actions.shape, jnp.int32),
    )(actions)


def _sc_gather(table, idx2d, n, d):
    n_w = n // _NW
    group = _CH * _GR
    n_groups = n_w // group

    mesh = plsc.VectorSubcoreMesh(core_axis_name="c", subcore_axis_name="s")

    @functools.partial(
        pl.kernel,
        out_type=jax.ShapeDtypeStruct((n, d), jnp.int32),
        mesh=mesh,
        scratch_types=[
            pltpu.VMEM((_GR, _CH), jnp.int32),
            pltpu.VMEM((group, d), jnp.int32),
            pltpu.SemaphoreType.DMA,
        ],
        compiler_params=pltpu.CompilerParams(use_tc_tiling_on_sc=False),
    )
    def k(table_hbm, idx_hbm, out_hbm, idx_v, rows_v, sem):
        wid = lax.axis_index("s") * _NC + lax.axis_index("c")
        base = wid * n_w

        def body(g, carry):
            gbase = pl.multiple_of(base + g * group, group)
            pltpu.sync_copy(
                idx_hbm.at[pl.ds(pl.multiple_of(gbase // _CH, _GR), _GR)], idx_v
            )
            copies = [
                pltpu.async_copy(
                    table_hbm.at[idx_v.at[j]],
                    rows_v.at[pl.ds(j * _CH, _CH)],
                    sem,
                )
                for j in range(_GR)
            ]
            for c in copies:
                c.wait()
            pltpu.sync_copy(rows_v, out_hbm.at[pl.ds(gbase, group)])
            return carry

        lax.fori_loop(0, n_groups, body, 0)

    return k(table, idx2d)


def kernel(actions, batch_time_shape, base_token, embedding):
    d = base_token.shape[0]
    if actions is None:
        batch_size, seq_len = batch_time_shape
        bt = base_token.astype(jnp.bfloat16)
        return jnp.broadcast_to(bt, (batch_size, seq_len, d))

    batch_size, seq_len = actions.shape
    v = embedding.shape[0]
    n = batch_size * seq_len

    block_rows = 1024
    v_pad = ((v + 1 + block_rows - 1) // block_rows) * block_rows

    table = _prep_table(base_token, embedding, v_pad, block_rows)
    table_i32 = lax.bitcast_convert_type(
        table.reshape(v_pad, d // 2, 2), jnp.int32
    )
    safe_idx = _remap_idx(jnp.asarray(actions, jnp.int32), v)
    idx2d = safe_idx.reshape(n // _CH, _CH)
    out = _sc_gather(table_i32, idx2d, n, d // 2)
    out_bf = lax.bitcast_convert_type(out, jnp.bfloat16)
    return out_bf.reshape(batch_size, seq_len, d)


# speedup vs baseline: 3.9804x; 2.9587x over previous
import functools

import jax
import jax.numpy as jnp
from jax import lax
from jax.experimental import pallas as pl
from jax.experimental.pallas import tpu as pltpu
from jax.experimental.pallas import tpu_sc as plsc

_NC = 2
_NS = 16
_NW = _NC * _NS

_CH = 128
_GR = 8


def _prep_table(base_token, emb_t, v, v_pad, cblk):
    d = emb_t.shape[0]
    half = d // 2

    def body(bt_ref, emb_ref, out_ref):
        i = pl.program_id(0)
        cols = jax.lax.broadcasted_iota(jnp.int32, (1, cblk), 1) + i * cblk
        e_bf = emb_ref[...].astype(jnp.bfloat16)
        bt_bf = bt_ref[...].astype(jnp.bfloat16)
        z = jnp.where(cols < v, e_bf, jnp.zeros((), jnp.bfloat16)) + bt_bf
        u = jax.lax.bitcast_convert_type(z.astype(jnp.float32), jnp.uint32)
        top = u & jnp.uint32(0xFFFF0000)
        word = (top[:half, :] >> 16) | top[half:, :]
        out_ref[...] = jax.lax.bitcast_convert_type(word, jnp.int32).T

    return pl.pallas_call(
        body,
        grid=(v_pad // cblk,),
        in_specs=[
            pl.BlockSpec((d, 1), lambda i: (0, 0)),
            pl.BlockSpec((d, cblk), lambda i: (0, i)),
        ],
        out_specs=pl.BlockSpec((cblk, half), lambda i: (i, 0)),
        out_shape=jax.ShapeDtypeStruct((v_pad, half), jnp.int32),
    )(base_token.reshape(d, 1), emb_t)


def _remap_idx(actions_t, v):
    def body(a_ref, o_ref):
        a = a_ref[...]
        o_ref[...] = jnp.where(a >= 0, a, jnp.int32(v))

    return pl.pallas_call(
        body,
        out_shape=jax.ShapeDtypeStruct(actions_t.shape, jnp.int32),
    )(actions_t)


def _sc_gather(table, idx2d, n, half):
    n_w = n // _NW
    group = _CH * _GR
    n_groups = n_w // group

    mesh = plsc.VectorSubcoreMesh(core_axis_name="c", subcore_axis_name="s")

    @functools.partial(
        pl.kernel,
        out_type=jax.ShapeDtypeStruct((n, half), jnp.int32),
        mesh=mesh,
        scratch_types=[
            pltpu.VMEM((_GR, _CH), jnp.int32),
            pltpu.VMEM((group, half), jnp.int32),
            pltpu.SemaphoreType.DMA,
        ],
        compiler_params=pltpu.CompilerParams(use_tc_tiling_on_sc=False),
    )
    def k(table_hbm, idx_hbm, out_hbm, idx_v, rows_v, sem):
        wid = lax.axis_index("s") * _NC + lax.axis_index("c")
        base = wid * n_w

        def body(g, carry):
            gbase = pl.multiple_of(base + g * group, group)
            pltpu.sync_copy(
                idx_hbm.at[pl.ds(pl.multiple_of(gbase // _CH, _GR), _GR)], idx_v
            )
            copies = [
                pltpu.async_copy(
                    table_hbm.at[idx_v.at[j]],
                    rows_v.at[pl.ds(j * _CH, _CH)],
                    sem,
                )
                for j in range(_GR)
            ]
            for c in copies:
                c.wait()
            pltpu.sync_copy(rows_v, out_hbm.at[pl.ds(gbase, group)])
            return carry

        lax.fori_loop(0, n_groups, body, 0)

    return k(table, idx2d)


def _unpack(packed3, bsz, seq, d, sblk, bblk):
    half = d // 2

    def body(x_ref, o_ref):
        u = jax.lax.bitcast_convert_type(x_ref[...], jnp.uint32)
        for s in range(sblk):
            wt = u[s].T
            lo = jax.lax.bitcast_convert_type(wt << 16, jnp.float32)
            hi = jax.lax.bitcast_convert_type(
                wt & jnp.uint32(0xFFFF0000), jnp.float32
            )
            o_ref[s, 0:half, :] = lo.astype(jnp.bfloat16)
            o_ref[s, half:d, :] = hi.astype(jnp.bfloat16)

    return pl.pallas_call(
        body,
        grid=(seq // sblk, bsz // bblk),
        in_specs=[pl.BlockSpec((sblk, bblk, half), lambda i, j: (i, j, 0))],
        out_specs=pl.BlockSpec((sblk, d, bblk), lambda i, j: (i, 0, j)),
        out_shape=jax.ShapeDtypeStruct((seq, d, bsz), jnp.bfloat16),
    )(packed3)


def kernel(actions, batch_time_shape, base_token, embedding):
    d = base_token.shape[0]
    if actions is None:
        batch_size, seq_len = batch_time_shape
        bt = base_token.astype(jnp.bfloat16)
        return jnp.broadcast_to(bt, (batch_size, seq_len, d))

    batch_size, seq_len = actions.shape
    v = embedding.shape[0]
    n = batch_size * seq_len
    half = d // 2

    cblk = 2048
    v_pad = ((v + 1 + cblk - 1) // cblk) * cblk

    table = _prep_table(base_token, embedding.T, v, v_pad, cblk)
    safe_idx = _remap_idx(jnp.asarray(actions, jnp.int32).T, v)
    idx2d = safe_idx.reshape(n // _CH, _CH)
    packed = _sc_gather(table, idx2d, n, half)
    packed3 = packed.reshape(seq_len, batch_size, half)
    out_t = _unpack(packed3, batch_size, seq_len, d, sblk=8, bblk=512)
    return jnp.transpose(out_t, (2, 0, 1))


# speedup vs baseline: 5.9805x; 1.5025x over previous
import functools

import jax
import jax.numpy as jnp
from jax import lax
from jax.experimental import pallas as pl
from jax.experimental.pallas import tpu as pltpu
from jax.experimental.pallas import tpu_sc as plsc

_NC = 2
_NS = 16
_NW = _NC * _NS

_CH = 128
_GR = 8


def _prep_table(base_token, emb_t, v, v_pad, pblk):
    d = emb_t.shape[0]
    quarter = v_pad // 4
    nblk = quarter // pblk

    def body(bt_ref, e0, e1, e2, e3, out_ref):
        i = pl.program_id(0)
        bt_bf = bt_ref[...].astype(jnp.bfloat16)
        for k, e_ref in enumerate((e0, e1, e2, e3)):
            rows = (
                jax.lax.broadcasted_iota(jnp.int32, (1, pblk), 1)
                + (k * quarter + i * pblk)
            )
            e_bf = e_ref[...].astype(jnp.bfloat16)
            z = jnp.where(rows < v, e_bf, jnp.zeros((), jnp.bfloat16)) + bt_bf
            w = pltpu.bitcast(z, jnp.int32)
            out_ref[:, k * (d // 2):(k + 1) * (d // 2)] = w.T

    def espec(k):
        return pl.BlockSpec((d, pblk), lambda i, k=k: (0, k * nblk + i))

    return pl.pallas_call(
        body,
        grid=(nblk,),
        in_specs=[
            pl.BlockSpec((d, 1), lambda i: (0, 0)),
            espec(0), espec(1), espec(2), espec(3),
        ],
        out_specs=pl.BlockSpec((pblk, 2 * d), lambda i: (i, 0)),
        out_shape=jax.ShapeDtypeStruct((quarter, 2 * d), jnp.int32),
    )(base_token.reshape(d, 1), emb_t, emb_t, emb_t, emb_t)


def _remap_idx(actions_t, v, quarter):
    def body(a_ref, o_ref):
        a = a_ref[...]
        safe = jnp.where(a >= 0, a, jnp.int32(v))
        o_ref[...] = (safe % 4) * quarter + safe // 4

    return pl.pallas_call(
        body,
        out_shape=jax.ShapeDtypeStruct(actions_t.shape, jnp.int32),
    )(actions_t)


def _sc_gather(table, idx2d, n, half):
    n_w = n // _NW
    group = _CH * _GR
    n_groups = n_w // group

    mesh = plsc.VectorSubcoreMesh(core_axis_name="c", subcore_axis_name="s")

    @functools.partial(
        pl.kernel,
        out_type=jax.ShapeDtypeStruct((n, half), jnp.int32),
        mesh=mesh,
        scratch_types=[
            pltpu.VMEM((_GR, _CH), jnp.int32),
            pltpu.VMEM((group, half), jnp.int32),
            pltpu.SemaphoreType.DMA,
        ],
        compiler_params=pltpu.CompilerParams(use_tc_tiling_on_sc=False),
    )
    def k(table_hbm, idx_hbm, out_hbm, idx_v, rows_v, sem):
        wid = lax.axis_index("s") * _NC + lax.axis_index("c")
        base = wid * n_w

        def body(g, carry):
            gbase = pl.multiple_of(base + g * group, group)
            pltpu.sync_copy(
                idx_hbm.at[pl.ds(pl.multiple_of(gbase // _CH, _GR), _GR)], idx_v
            )
            copies = [
                pltpu.async_copy(
                    table_hbm.at[idx_v.at[j]],
                    rows_v.at[pl.ds(j * _CH, _CH)],
                    sem,
                )
                for j in range(_GR)
            ]
            for c in copies:
                c.wait()
            pltpu.sync_copy(rows_v, out_hbm.at[pl.ds(gbase, group)])
            return carry

        lax.fori_loop(0, n_groups, body, 0)

    return k(table, idx2d)


def _unpack(packed128, bsz, seq, d, sblk):
    half = d // 2
    rps = bsz * half // 128
    qg = 128 // half

    def body(x_ref, o_ref):
        for s in range(sblk):
            x = x_ref[pl.ds(s * rps, rps), :]
            w = jnp.concatenate(
                [x[:, k * half:(k + 1) * half].T for k in range(qg)], axis=1
            )
            o_ref[s] = pltpu.bitcast(w, jnp.bfloat16)

    return pl.pallas_call(
        body,
        grid=(seq // sblk,),
        in_specs=[pl.BlockSpec((sblk * rps, 128), lambda i: (i, 0))],
        out_specs=pl.BlockSpec((sblk, d, bsz), lambda i: (i, 0, 0)),
        out_shape=jax.ShapeDtypeStruct((seq, d, bsz), jnp.bfloat16),
    )(packed128)


def kernel(actions, batch_time_shape, base_token, embedding):
    d = base_token.shape[0]
    if actions is None:
        batch_size, seq_len = batch_time_shape
        bt = base_token.astype(jnp.bfloat16)
        return jnp.broadcast_to(bt, (batch_size, seq_len, d))

    batch_size, seq_len = actions.shape
    v = embedding.shape[0]
    n = batch_size * seq_len
    half = d // 2

    v_pad = 100352
    quarter = v_pad // 4

    table128 = _prep_table(base_token, embedding.T, v, v_pad, pblk=3584)
    table = table128.reshape(v_pad, half)

    safe_idx = _remap_idx(jnp.asarray(actions, jnp.int32).T, v, quarter)
    bq = batch_size // 4
    idx_perm = (
        safe_idx.reshape(seq_len, 4, bq)
        .transpose(0, 2, 1)
        .reshape(seq_len, batch_size)
    )
    idx2d = idx_perm.reshape(n // _CH, _CH)

    packed = _sc_gather(table, idx2d, n, half)
    packed128 = packed.reshape(n * half // 128, 128)
    out_t = _unpack(packed128, batch_size, seq_len, d, sblk=2)
    return jnp.transpose(out_t, (2, 0, 1))


# speedup vs baseline: 5.9858x; 1.0009x over previous
import functools

import jax
import jax.numpy as jnp
from jax import lax
from jax.experimental import pallas as pl
from jax.experimental.pallas import tpu as pltpu
from jax.experimental.pallas import tpu_sc as plsc

_NC = 2
_NS = 16
_NW = _NC * _NS

_CH = 128
_GR = 8


def _prep_table(base_token, emb_t, v, v_pad, pblk):
    d = emb_t.shape[0]
    quarter = v_pad // 4
    nblk = quarter // pblk

    def body(bt_ref, e0, e1, e2, e3, out_ref):
        i = pl.program_id(0)
        bt_bf = bt_ref[...].astype(jnp.bfloat16)
        for k, e_ref in enumerate((e0, e1, e2, e3)):
            rows = (
                jax.lax.broadcasted_iota(jnp.int32, (1, pblk), 1)
                + (k * quarter + i * pblk)
            )
            e_bf = e_ref[...].astype(jnp.bfloat16)
            z = jnp.where(rows < v, e_bf, jnp.zeros((), jnp.bfloat16)) + bt_bf
            w = pltpu.bitcast(z, jnp.int32)
            out_ref[:, k * (d // 2):(k + 1) * (d // 2)] = w.T

    def espec(k):
        return pl.BlockSpec((d, pblk), lambda i, k=k: (0, k * nblk + i))

    return pl.pallas_call(
        body,
        grid=(nblk,),
        in_specs=[
            pl.BlockSpec((d, 1), lambda i: (0, 0)),
            espec(0), espec(1), espec(2), espec(3),
        ],
        out_specs=pl.BlockSpec((pblk, 2 * d), lambda i: (i, 0)),
        out_shape=jax.ShapeDtypeStruct((quarter, 2 * d), jnp.int32),
    )(base_token.reshape(d, 1), emb_t, emb_t, emb_t, emb_t)


def _remap_idx(actions_t, v, quarter):
    def body(a_ref, o_ref):
        a = a_ref[...]
        safe = jnp.where(a >= 0, a, jnp.int32(v))
        o_ref[...] = (safe % quarter) * 4 + safe // quarter

    return pl.pallas_call(
        body,
        out_shape=jax.ShapeDtypeStruct(actions_t.shape, jnp.int32),
    )(actions_t)


def _sc_gather(table, idx2d, n, half):
    n_w = n // _NW
    group = _CH * _GR
    n_groups = n_w // group

    mesh = plsc.VectorSubcoreMesh(core_axis_name="c", subcore_axis_name="s")

    @functools.partial(
        pl.kernel,
        out_type=jax.ShapeDtypeStruct((n, half), jnp.int32),
        mesh=mesh,
        scratch_types=[
            pltpu.VMEM((_GR, _CH), jnp.int32),
            pltpu.VMEM((group, half), jnp.int32),
            pltpu.SemaphoreType.DMA,
        ],
        compiler_params=pltpu.CompilerParams(use_tc_tiling_on_sc=False),
    )
    def k(table_hbm, idx_hbm, out_hbm, idx_v, rows_v, sem):
        wid = lax.axis_index("s") * _NC + lax.axis_index("c")
        base = wid * n_w

        def body(g, carry):
            gbase = pl.multiple_of(base + g * group, group)
            pltpu.sync_copy(
                idx_hbm.at[pl.ds(pl.multiple_of(gbase // _CH, _GR), _GR)], idx_v
            )
            copies = [
                pltpu.async_copy(
                    table_hbm.at[idx_v.at[j]],
                    rows_v.at[pl.ds(j * _CH, _CH)],
                    sem,
                )
                for j in range(_GR)
            ]
            for c in copies:
                c.wait()
            pltpu.sync_copy(rows_v, out_hbm.at[pl.ds(gbase, group)])
            return carry

        lax.fori_loop(0, n_groups, body, 0)

    return k(table, idx2d)


def _unpack(packed128, bsz, seq, d, sblk):
    half = d // 2
    rps = bsz * half // 128
    qg = 128 // half

    def body(x_ref, o_ref):
        for s in range(sblk):
            x = x_ref[pl.ds(s * rps, rps), :]
            w = jnp.concatenate(
                [x[:, k * half:(k + 1) * half].T for k in range(qg)], axis=1
            )
            o_ref[s] = pltpu.bitcast(w, jnp.bfloat16)

    return pl.pallas_call(
        body,
        grid=(seq // sblk,),
        in_specs=[pl.BlockSpec((sblk * rps, 128), lambda i: (i, 0))],
        out_specs=pl.BlockSpec((sblk, d, bsz), lambda i: (i, 0, 0)),
        out_shape=jax.ShapeDtypeStruct((seq, d, bsz), jnp.bfloat16),
    )(packed128)


def kernel(actions, batch_time_shape, base_token, embedding):
    d = base_token.shape[0]
    if actions is None:
        batch_size, seq_len = batch_time_shape
        bt = base_token.astype(jnp.bfloat16)
        return jnp.broadcast_to(bt, (batch_size, seq_len, d))

    batch_size, seq_len = actions.shape
    v = embedding.shape[0]
    n = batch_size * seq_len
    half = d // 2

    v_pad = 100352
    quarter = v_pad // 4

    table128 = _prep_table(base_token, embedding.T, v, v_pad, pblk=3584)
    table = table128.reshape(v_pad, half)

    safe_idx = _remap_idx(jnp.asarray(actions, jnp.int32).T, v, quarter)
    bq = batch_size // 4
    idx_perm = (
        safe_idx.reshape(seq_len, 4, bq)
        .transpose(0, 2, 1)
        .reshape(seq_len, batch_size)
    )
    idx2d = idx_perm.reshape(n // _CH, _CH)

    packed = _sc_gather(table, idx2d, n, half)
    packed128 = packed.reshape(n * half // 128, 128)
    out_t = _unpack(packed128, batch_size, seq_len, d, sblk=2)
    return jnp.transpose(out_t, (2, 0, 1))


# speedup vs baseline: 7.0147x; 1.1719x over previous
import functools

import jax
import jax.numpy as jnp
from jax import lax
from jax.experimental import pallas as pl
from jax.experimental.pallas import tpu as pltpu
from jax.experimental.pallas import tpu_sc as plsc

_NC = 2
_NS = 16
_NW = _NC * _NS

_CH = 128
_GR = 8


def _prep_table(base_token, emb_t, v, v_pad, pblk):
    d = emb_t.shape[0]
    quarter = v_pad // 4
    nblk = quarter // pblk

    def body(bt_ref, e0, e1, e2, e3, out_ref):
        i = pl.program_id(0)
        bt_bf = bt_ref[...].astype(jnp.bfloat16)
        for k, e_ref in enumerate((e0, e1, e2, e3)):
            rows = (
                jax.lax.broadcasted_iota(jnp.int32, (1, pblk), 1)
                + (k * quarter + i * pblk)
            )
            e_bf = e_ref[...].astype(jnp.bfloat16)
            z = jnp.where(rows < v, e_bf, jnp.zeros((), jnp.bfloat16)) + bt_bf
            w = pltpu.bitcast(z, jnp.int32)
            out_ref[:, k * (d // 2):(k + 1) * (d // 2)] = w.T

    def espec(k):
        return pl.BlockSpec((d, pblk), lambda i, k=k: (0, k * nblk + i))

    return pl.pallas_call(
        body,
        grid=(nblk,),
        in_specs=[
            pl.BlockSpec((d, 1), lambda i: (0, 0)),
            espec(0), espec(1), espec(2), espec(3),
        ],
        out_specs=pl.BlockSpec((pblk, 2 * d), lambda i: (i, 0)),
        out_shape=jax.ShapeDtypeStruct((quarter, 2 * d), jnp.int32),
    )(base_token.reshape(d, 1), emb_t, emb_t, emb_t, emb_t)


def _remap_idx(actions_t, v, quarter):
    def body(a_ref, o_ref):
        a = a_ref[...]
        safe = jnp.where(a >= 0, a, jnp.int32(v))
        m = (safe % quarter) * 4 + safe // quarter
        o_ref[...] = pltpu.einshape("s(kp)->s(pk)", m, k=4)

    seq, bsz = actions_t.shape
    sblk = 8
    return pl.pallas_call(
        body,
        grid=(seq // sblk,),
        in_specs=[pl.BlockSpec((sblk, bsz), lambda i: (i, 0))],
        out_specs=pl.BlockSpec((sblk, bsz), lambda i: (i, 0)),
        out_shape=jax.ShapeDtypeStruct(actions_t.shape, jnp.int32),
    )(actions_t)


def _sc_gather(table, idx2d, n, half):
    n_w = n // _NW
    group = _CH * _GR
    n_groups = n_w // group

    mesh = plsc.VectorSubcoreMesh(core_axis_name="c", subcore_axis_name="s")

    @functools.partial(
        pl.kernel,
        out_type=jax.ShapeDtypeStruct((n, half), jnp.int32),
        mesh=mesh,
        scratch_types=[
            pltpu.VMEM((_GR, _CH), jnp.int32),
            pltpu.VMEM((group, half), jnp.int32),
            pltpu.SemaphoreType.DMA,
        ],
        compiler_params=pltpu.CompilerParams(use_tc_tiling_on_sc=False),
    )
    def k(table_hbm, idx_hbm, out_hbm, idx_v, rows_v, sem):
        wid = lax.axis_index("s") * _NC + lax.axis_index("c")
        base = wid * n_w

        def body(g, carry):
            gbase = pl.multiple_of(base + g * group, group)
            pltpu.sync_copy(
                idx_hbm.at[pl.ds(pl.multiple_of(gbase // _CH, _GR), _GR)], idx_v
            )
            copies = [
                pltpu.async_copy(
                    table_hbm.at[idx_v.at[j]],
                    rows_v.at[pl.ds(j * _CH, _CH)],
                    sem,
                )
                for j in range(_GR)
            ]
            for c in copies:
                c.wait()
            pltpu.sync_copy(rows_v, out_hbm.at[pl.ds(gbase, group)])
            return carry

        lax.fori_loop(0, n_groups, body, 0)

    return k(table, idx2d)


def _unpack(packed128, bsz, seq, d, sblk):
    half = d // 2
    rps = bsz * half // 128
    qg = 128 // half

    def body(x_ref, o_ref):
        for s in range(sblk):
            x = x_ref[pl.ds(s * rps, rps), :]
            w = jnp.concatenate(
                [x[:, k * half:(k + 1) * half].T for k in range(qg)], axis=1
            )
            o_ref[s] = pltpu.bitcast(w, jnp.bfloat16)

    return pl.pallas_call(
        body,
        grid=(seq // sblk,),
        in_specs=[pl.BlockSpec((sblk * rps, 128), lambda i: (i, 0))],
        out_specs=pl.BlockSpec((sblk, d, bsz), lambda i: (i, 0, 0)),
        out_shape=jax.ShapeDtypeStruct((seq, d, bsz), jnp.bfloat16),
    )(packed128)


def kernel(actions, batch_time_shape, base_token, embedding):
    d = base_token.shape[0]
    if actions is None:
        batch_size, seq_len = batch_time_shape
        bt = base_token.astype(jnp.bfloat16)
        return jnp.broadcast_to(bt, (batch_size, seq_len, d))

    batch_size, seq_len = actions.shape
    v = embedding.shape[0]
    n = batch_size * seq_len
    half = d // 2

    v_pad = 100352
    quarter = v_pad // 4

    table128 = _prep_table(base_token, embedding.T, v, v_pad, pblk=3584)
    table = table128.reshape(v_pad, half)

    idx_perm = _remap_idx(jnp.asarray(actions, jnp.int32).T, v, quarter)
    idx2d = idx_perm.reshape(n // _CH, _CH)

    packed = _sc_gather(table, idx2d, n, half)
    packed128 = packed.reshape(n * half // 128, 128)
    out_t = _unpack(packed128, batch_size, seq_len, d, sblk=2)
    return jnp.transpose(out_t, (2, 0, 1))


# speedup vs baseline: 8.8601x; 1.2631x over previous
import functools

import jax
import jax.numpy as jnp
from jax import lax
from jax.experimental import pallas as pl
from jax.experimental.pallas import tpu as pltpu
from jax.experimental.pallas import tpu_sc as plsc

_NC = 2
_NS = 16
_NW = _NC * _NS

_CH = 128
_GR = 8


def _prep_table(base_token, emb_t, v, v_pad, pblk):
    d = emb_t.shape[0]
    quarter = v_pad // 4
    nblk = quarter // pblk

    def body(bt_ref, e0, e1, e2, e3, out_ref):
        i = pl.program_id(0)
        bt_bf = bt_ref[...].astype(jnp.bfloat16)
        for k, e_ref in enumerate((e0, e1, e2, e3)):
            rows = (
                jax.lax.broadcasted_iota(jnp.int32, (1, pblk), 1)
                + (k * quarter + i * pblk)
            )
            e_bf = e_ref[...].astype(jnp.bfloat16)
            z = jnp.where(rows < v, e_bf, jnp.zeros((), jnp.bfloat16)) + bt_bf
            w = pltpu.bitcast(z, jnp.int32)
            out_ref[:, k * (d // 2):(k + 1) * (d // 2)] = w.T

    def espec(k):
        return pl.BlockSpec((d, pblk), lambda i, k=k: (0, k * nblk + i))

    return pl.pallas_call(
        body,
        grid=(nblk,),
        in_specs=[
            pl.BlockSpec((d, 1), lambda i: (0, 0)),
            espec(0), espec(1), espec(2), espec(3),
        ],
        out_specs=pl.BlockSpec((pblk, 2 * d), lambda i: (i, 0)),
        out_shape=jax.ShapeDtypeStruct((quarter, 2 * d), jnp.int32),
    )(base_token.reshape(d, 1), emb_t, emb_t, emb_t, emb_t)


def _remap_idx(actions_t, v, quarter):
    def body(a_ref, o_ref):
        a = a_ref[...]
        safe = jnp.where(a >= 0, a, jnp.int32(v))
        m = (safe % quarter) * 4 + safe // quarter
        o_ref[...] = pltpu.einshape("s(kp)->s(pk)", m, k=4)

    seq, bsz = actions_t.shape
    sblk = 8
    return pl.pallas_call(
        body,
        grid=(seq // sblk,),
        in_specs=[pl.BlockSpec((sblk, bsz), lambda i: (i, 0))],
        out_specs=pl.BlockSpec((sblk, bsz), lambda i: (i, 0)),
        out_shape=jax.ShapeDtypeStruct(actions_t.shape, jnp.int32),
    )(actions_t)


def _sc_gather(table, idx2d, n, half):
    n_w = n // _NW
    group = _CH * _GR
    n_groups = n_w // group

    mesh = plsc.VectorSubcoreMesh(core_axis_name="c", subcore_axis_name="s")

    @functools.partial(
        pl.kernel,
        out_type=jax.ShapeDtypeStruct((n, half), jnp.int32),
        mesh=mesh,
        scratch_types=[
            pltpu.VMEM((_GR, _CH), jnp.int32),
            pltpu.VMEM((group, half), jnp.int32),
            pltpu.SemaphoreType.DMA,
        ],
        compiler_params=pltpu.CompilerParams(use_tc_tiling_on_sc=False),
    )
    def k(table_hbm, idx_hbm, out_hbm, idx_v, rows_v, sem):
        wid = lax.axis_index("s") * _NC + lax.axis_index("c")
        base = wid * n_w

        def body(g, carry):
            gbase = pl.multiple_of(base + g * group, group)
            pltpu.sync_copy(
                idx_hbm.at[pl.ds(pl.multiple_of(gbase // _CH, _GR), _GR)], idx_v
            )
            copies = [
                pltpu.async_copy(
                    table_hbm.at[idx_v.at[j]],
                    rows_v.at[pl.ds(j * _CH, _CH)],
                    sem,
                )
                for j in range(_GR)
            ]
            for c in copies:
                c.wait()
            pltpu.sync_copy(rows_v, out_hbm.at[pl.ds(gbase, group)])
            return carry

        lax.fori_loop(0, n_groups, body, 0)

    return k(table, idx2d)


def _unpack(packed128, bsz, seq, d, sblk):
    half = d // 2
    rps = bsz * half // 128
    qg = 128 // half

    bq = bsz // qg

    def body(x_ref, o_ref):
        for s in range(sblk):
            xt = x_ref[pl.ds(s * rps, rps), :].T
            for k in range(qg):
                w = xt[k * half:(k + 1) * half, :]
                o_ref[s, :, pl.ds(k * bq, bq)] = pltpu.bitcast(w, jnp.bfloat16)

    return pl.pallas_call(
        body,
        grid=(seq // sblk,),
        in_specs=[pl.BlockSpec((sblk * rps, 128), lambda i: (i, 0))],
        out_specs=pl.BlockSpec((sblk, d, bsz), lambda i: (i, 0, 0)),
        out_shape=jax.ShapeDtypeStruct((seq, d, bsz), jnp.bfloat16),
    )(packed128)


def kernel(actions, batch_time_shape, base_token, embedding):
    d = base_token.shape[0]
    if actions is None:
        batch_size, seq_len = batch_time_shape
        bt = base_token.astype(jnp.bfloat16)
        return jnp.broadcast_to(bt, (batch_size, seq_len, d))

    batch_size, seq_len = actions.shape
    v = embedding.shape[0]
    n = batch_size * seq_len
    half = d // 2

    v_pad = 100352
    quarter = v_pad // 4

    table128 = _prep_table(base_token, embedding.T, v, v_pad, pblk=3584)
    table = table128.reshape(v_pad, half)

    idx_perm = _remap_idx(jnp.asarray(actions, jnp.int32).T, v, quarter)
    idx2d = idx_perm.reshape(n // _CH, _CH)

    packed = _sc_gather(table, idx2d, n, half)
    packed128 = packed.reshape(n * half // 128, 128)
    out_t = _unpack(packed128, batch_size, seq_len, d, sblk=2)
    return jnp.transpose(out_t, (2, 0, 1))


# speedup vs baseline: 9.3458x; 1.0548x over previous
import functools

import jax
import jax.numpy as jnp
from jax import lax
from jax.experimental import pallas as pl
from jax.experimental.pallas import tpu as pltpu
from jax.experimental.pallas import tpu_sc as plsc

_NC = 2
_NS = 16
_NW = _NC * _NS

_CH = 128
_GR = 8


def _prep_table(base_token, emb_t, v, v_pad, pblk):
    d = emb_t.shape[0]
    quarter = v_pad // 4
    nblk = quarter // pblk

    def body(bt_ref, e0, e1, e2, e3, out_ref):
        i = pl.program_id(0)
        bt_bf = bt_ref[...].astype(jnp.bfloat16)
        for k, e_ref in enumerate((e0, e1, e2, e3)):
            rows = (
                jax.lax.broadcasted_iota(jnp.int32, (1, pblk), 1)
                + (k * quarter + i * pblk)
            )
            e_bf = e_ref[...].astype(jnp.bfloat16)
            z = jnp.where(rows < v, e_bf, jnp.zeros((), jnp.bfloat16)) + bt_bf
            w = pltpu.bitcast(z, jnp.int32)
            out_ref[:, k * (d // 2):(k + 1) * (d // 2)] = w.T

    def espec(k):
        return pl.BlockSpec((d, pblk), lambda i, k=k: (0, k * nblk + i))

    return pl.pallas_call(
        body,
        grid=(nblk,),
        in_specs=[
            pl.BlockSpec((d, 1), lambda i: (0, 0)),
            espec(0), espec(1), espec(2), espec(3),
        ],
        out_specs=pl.BlockSpec((pblk, 2 * d), lambda i: (i, 0)),
        out_shape=jax.ShapeDtypeStruct((quarter, 2 * d), jnp.int32),
    )(base_token.reshape(d, 1), emb_t, emb_t, emb_t, emb_t)


def _remap_idx(actions_t, v, quarter):
    def body(a_ref, o_ref):
        a = a_ref[...]
        safe = jnp.where(a >= 0, a, jnp.int32(v))
        m = (safe % quarter) * 4 + safe // quarter
        o_ref[...] = pltpu.einshape("s(kp)->s(pk)", m, k=4)

    seq, bsz = actions_t.shape
    sblk = 8
    return pl.pallas_call(
        body,
        grid=(seq // sblk,),
        in_specs=[pl.BlockSpec((sblk, bsz), lambda i: (i, 0))],
        out_specs=pl.BlockSpec((sblk, bsz), lambda i: (i, 0)),
        out_shape=jax.ShapeDtypeStruct(actions_t.shape, jnp.int32),
    )(actions_t)


def _sc_gather(table, idx2d, n, half):
    n_w = n // _NW
    group = _CH * _GR
    n_groups = n_w // group

    mesh = plsc.VectorSubcoreMesh(core_axis_name="c", subcore_axis_name="s")

    @functools.partial(
        pl.kernel,
        out_type=jax.ShapeDtypeStruct((n, half), jnp.int32),
        mesh=mesh,
        scratch_types=[
            pltpu.VMEM((2, _GR, _CH), jnp.int32),
            pltpu.VMEM((2, group, half), jnp.int32),
            pltpu.SemaphoreType.DMA,
            pltpu.SemaphoreType.DMA,
            pltpu.SemaphoreType.DMA,
            pltpu.SemaphoreType.DMA,
        ],
        compiler_params=pltpu.CompilerParams(use_tc_tiling_on_sc=False),
    )
    def k(table_hbm, idx_hbm, out_hbm, idx_v, rows_v, sg0, sg1, sw0, sw1):
        wid = lax.axis_index("s") * _NC + lax.axis_index("c")
        base = wid * n_w
        sg = (sg0, sg1)
        sw = (sw0, sw1)

        def gbase_of(g):
            return pl.multiple_of(base + g * group, group)

        def stage_fire(g, p):
            gbase = gbase_of(g)
            pltpu.sync_copy(
                idx_hbm.at[pl.ds(pl.multiple_of(gbase // _CH, _GR), _GR)],
                idx_v.at[p],
            )
            for j in range(_GR):
                pltpu.async_copy(
                    table_hbm.at[idx_v.at[p].at[j]],
                    rows_v.at[p].at[pl.ds(j * _CH, _CH)],
                    sg[p],
                )

        def drain_gathers(p):
            for j in range(_GR):
                pltpu.make_async_copy(
                    table_hbm.at[idx_v.at[p].at[j]],
                    rows_v.at[p].at[pl.ds(j * _CH, _CH)],
                    sg[p],
                ).wait()

        def wb_async(g, p):
            pltpu.async_copy(rows_v.at[p], out_hbm.at[pl.ds(gbase_of(g), group)], sw[p])

        def wb_wait(g, p):
            pltpu.make_async_copy(
                rows_v.at[p], out_hbm.at[pl.ds(gbase_of(g), group)], sw[p]
            ).wait()

        stage_fire(0, 0)

        def pair(i, carry):
            for j in (0, 1):
                g = 2 * i + 1 + j
                p = (1 + j) % 2
                q = 1 - p
                if j == 0:
                    @pl.when(i > 0)
                    def _():
                        wb_wait(g - 2, p)
                else:
                    wb_wait(g - 2, p)
                stage_fire(g, p)
                drain_gathers(q)
                wb_async(g - 1, q)
            return carry

        lax.fori_loop(0, (n_groups - 1) // 2, pair, 0)

        last = n_groups - 1
        drain_gathers(last % 2)
        wb_wait(last - 1, 1 - last % 2)
        pltpu.sync_copy(
            rows_v.at[last % 2], out_hbm.at[pl.ds(gbase_of(last), group)]
        )

    return k(table, idx2d)


def _unpack(packed128, bsz, seq, d, sblk):
    half = d // 2
    rps = bsz * half // 128
    qg = 128 // half

    bq = bsz // qg

    def body(x_ref, o_ref):
        for s in range(sblk):
            xt = x_ref[pl.ds(s * rps, rps), :].T
            for k in range(qg):
                w = xt[k * half:(k + 1) * half, :]
                o_ref[s, :, pl.ds(k * bq, bq)] = pltpu.bitcast(w, jnp.bfloat16)

    return pl.pallas_call(
        body,
        grid=(seq // sblk,),
        in_specs=[pl.BlockSpec((sblk * rps, 128), lambda i: (i, 0))],
        out_specs=pl.BlockSpec((sblk, d, bsz), lambda i: (i, 0, 0)),
        out_shape=jax.ShapeDtypeStruct((seq, d, bsz), jnp.bfloat16),
    )(packed128)


def kernel(actions, batch_time_shape, base_token, embedding):
    d = base_token.shape[0]
    if actions is None:
        batch_size, seq_len = batch_time_shape
        bt = base_token.astype(jnp.bfloat16)
        return jnp.broadcast_to(bt, (batch_size, seq_len, d))

    batch_size, seq_len = actions.shape
    v = embedding.shape[0]
    n = batch_size * seq_len
    half = d // 2

    v_pad = 100352
    quarter = v_pad // 4

    table128 = _prep_table(base_token, embedding.T, v, v_pad, pblk=3584)
    table = table128.reshape(v_pad, half)

    idx_perm = _remap_idx(jnp.asarray(actions, jnp.int32).T, v, quarter)
    idx2d = idx_perm.reshape(n // _CH, _CH)

    packed = _sc_gather(table, idx2d, n, half)
    packed128 = packed.reshape(n * half // 128, 128)
    out_t = _unpack(packed128, batch_size, seq_len, d, sblk=2)
    return jnp.transpose(out_t, (2, 0, 1))


# speedup vs baseline: 10.3803x; 1.1107x over previous
import functools

import jax
import jax.numpy as jnp
from jax import lax
from jax.experimental import pallas as pl
from jax.experimental.pallas import tpu as pltpu
from jax.experimental.pallas import tpu_sc as plsc

_NC = 2
_NS = 16
_NW = _NC * _NS

_CH = 128
_GR = 8


def _prep_table(base_token, emb_t, v, v_pad, pblk):
    d = emb_t.shape[0]
    quarter = v_pad // 4
    nblk = quarter // pblk

    def body(bt_ref, e0, e1, e2, e3, out_ref):
        i = pl.program_id(0)
        bt_bf = bt_ref[...].astype(jnp.bfloat16)
        for k, e_ref in enumerate((e0, e1, e2, e3)):
            rows = (
                jax.lax.broadcasted_iota(jnp.int32, (1, pblk), 1)
                + (k * quarter + i * pblk)
            )
            e_bf = e_ref[...].astype(jnp.bfloat16)
            z = jnp.where(rows < v, e_bf, jnp.zeros((), jnp.bfloat16)) + bt_bf
            w = pltpu.bitcast(z, jnp.int32)
            out_ref[:, k * (d // 2):(k + 1) * (d // 2)] = w.T

    def espec(k):
        return pl.BlockSpec((d, pblk), lambda i, k=k: (0, k * nblk + i))

    return pl.pallas_call(
        body,
        grid=(nblk,),
        in_specs=[
            pl.BlockSpec((d, 1), lambda i: (0, 0)),
            espec(0), espec(1), espec(2), espec(3),
        ],
        out_specs=pl.BlockSpec((pblk, 2 * d), lambda i: (i, 0)),
        out_shape=jax.ShapeDtypeStruct((quarter, 2 * d), jnp.int32),
    )(base_token.reshape(d, 1), emb_t, emb_t, emb_t, emb_t)


def _remap_idx(actions_t, v, quarter):
    def body(a_ref, o_ref):
        a = a_ref[...]
        safe = jnp.where(a >= 0, a, jnp.int32(v))
        m = (safe % quarter) * 4 + safe // quarter
        o_ref[...] = pltpu.einshape("s(kp)->s(pk)", m, k=4)

    seq, bsz = actions_t.shape
    sblk = 8
    return pl.pallas_call(
        body,
        grid=(seq // sblk,),
        in_specs=[pl.BlockSpec((sblk, bsz), lambda i: (i, 0))],
        out_specs=pl.BlockSpec((sblk, bsz), lambda i: (i, 0)),
        out_shape=jax.ShapeDtypeStruct(actions_t.shape, jnp.int32),
    )(actions_t)


def _sc_gather(table, idx2d, n, half):
    n_w = n // _NW
    group = _CH * _GR
    n_groups = n_w // group

    mesh = plsc.VectorSubcoreMesh(core_axis_name="c", subcore_axis_name="s")

    @functools.partial(
        pl.kernel,
        out_type=jax.ShapeDtypeStruct((n, half), jnp.int32),
        mesh=mesh,
        scratch_types=[
            pltpu.VMEM((2, _GR, _CH), jnp.int32),
            pltpu.VMEM((2, group, half), jnp.int32),
            pltpu.SemaphoreType.DMA,
            pltpu.SemaphoreType.DMA,
            pltpu.SemaphoreType.DMA,
            pltpu.SemaphoreType.DMA,
        ],
        compiler_params=pltpu.CompilerParams(use_tc_tiling_on_sc=False),
    )
    def k(table_hbm, idx_hbm, out_hbm, idx_v, rows_v, sg0, sg1, sw0, sw1):
        wid = lax.axis_index("s") * _NC + lax.axis_index("c")
        base = wid * n_w
        sg = (sg0, sg1)
        sw = (sw0, sw1)

        def gbase_of(g):
            return pl.multiple_of(base + g * group, group)

        def stage_fire(g, p):
            gbase = gbase_of(g)
            pltpu.sync_copy(
                idx_hbm.at[pl.ds(pl.multiple_of(gbase // _CH, _GR), _GR)],
                idx_v.at[p],
            )
            for j in range(_GR):
                pltpu.async_copy(
                    table_hbm.at[idx_v.at[p].at[j]],
                    rows_v.at[p].at[pl.ds(j * _CH, _CH)],
                    sg[p],
                )

        def drain_gathers(p):
            for j in range(_GR):
                pltpu.make_async_copy(
                    table_hbm.at[idx_v.at[p].at[j]],
                    rows_v.at[p].at[pl.ds(j * _CH, _CH)],
                    sg[p],
                ).wait()

        def wb_async(g, p):
            pltpu.async_copy(rows_v.at[p], out_hbm.at[pl.ds(gbase_of(g), group)], sw[p])

        def wb_wait(g, p):
            pltpu.make_async_copy(
                rows_v.at[p], out_hbm.at[pl.ds(gbase_of(g), group)], sw[p]
            ).wait()

        stage_fire(0, 0)

        def pair(i, carry):
            for j in (0, 1):
                g = 2 * i + 1 + j
                p = (1 + j) % 2
                q = 1 - p
                if j == 0:
                    @pl.when(i > 0)
                    def _():
                        wb_wait(g - 2, p)
                else:
                    wb_wait(g - 2, p)
                stage_fire(g, p)
                drain_gathers(q)
                wb_async(g - 1, q)
            return carry

        lax.fori_loop(0, (n_groups - 1) // 2, pair, 0)

        last = n_groups - 1
        drain_gathers(last % 2)
        wb_wait(last - 1, 1 - last % 2)
        pltpu.sync_copy(
            rows_v.at[last % 2], out_hbm.at[pl.ds(gbase_of(last), group)]
        )

    return k(table, idx2d)


def _unpack(packed128, bsz, seq, d, sblk):
    half = d // 2
    rps = bsz * half // 128
    qg = 128 // half

    bq = bsz // qg

    def body(x_ref, o_ref):
        for s in range(sblk):
            xt = x_ref[pl.ds(s * rps, rps), :].T
            for k in range(qg):
                w = xt[k * half:(k + 1) * half, :]
                o_ref[s, :, pl.ds(k * bq, bq)] = pltpu.bitcast(w, jnp.bfloat16)

    return pl.pallas_call(
        body,
        grid=(seq // sblk,),
        in_specs=[pl.BlockSpec((sblk * rps, 128), lambda i: (i, 0))],
        out_specs=pl.BlockSpec((sblk, d, bsz), lambda i: (i, 0, 0)),
        out_shape=jax.ShapeDtypeStruct((seq, d, bsz), jnp.bfloat16),
    )(packed128)


def kernel(actions, batch_time_shape, base_token, embedding):
    d = base_token.shape[0]
    if actions is None:
        batch_size, seq_len = batch_time_shape
        bt = base_token.astype(jnp.bfloat16)
        return jnp.broadcast_to(bt, (batch_size, seq_len, d))

    batch_size, seq_len = actions.shape
    v = embedding.shape[0]
    n = batch_size * seq_len
    half = d // 2

    v_pad = 100352
    quarter = v_pad // 4

    table128 = _prep_table(base_token, embedding.T, v, v_pad, pblk=3584)
    table = table128.reshape(v_pad, half)

    idx_perm = _remap_idx(jnp.asarray(actions, jnp.int32).T, v, quarter)
    idx2d = idx_perm.reshape(n // _CH, _CH)

    packed = _sc_gather(table, idx2d, n, half)
    packed128 = packed.reshape(n * half // 128, 128)
    out_t = _unpack(packed128, batch_size, seq_len, d, sblk=4)
    return jnp.transpose(out_t, (2, 0, 1))


# speedup vs baseline: 10.8443x; 1.0447x over previous
import functools

import jax
import jax.numpy as jnp
from jax import lax
from jax.experimental import pallas as pl
from jax.experimental.pallas import tpu as pltpu
from jax.experimental.pallas import tpu_sc as plsc

_NC = 2
_NS = 16
_NW = _NC * _NS

_CH = 128
_GR = 8


def _prep_table(base_token, emb_t, v, v_pad, pblk):
    d = emb_t.shape[0]
    quarter = v_pad // 4
    nblk = quarter // pblk

    def body(bt_ref, e0, e1, e2, e3, out_ref):
        i = pl.program_id(0)
        bt_bf = bt_ref[...].astype(jnp.bfloat16)
        for k, e_ref in enumerate((e0, e1, e2, e3)):
            rows = (
                jax.lax.broadcasted_iota(jnp.int32, (1, pblk), 1)
                + (k * quarter + i * pblk)
            )
            e_bf = e_ref[...].astype(jnp.bfloat16)
            z = jnp.where(rows < v, e_bf, jnp.zeros((), jnp.bfloat16)) + bt_bf
            w = pltpu.bitcast(z, jnp.int32)
            out_ref[:, k * (d // 2):(k + 1) * (d // 2)] = w.T

    def espec(k):
        return pl.BlockSpec((d, pblk), lambda i, k=k: (0, k * nblk + i))

    return pl.pallas_call(
        body,
        grid=(nblk,),
        in_specs=[
            pl.BlockSpec((d, 1), lambda i: (0, 0)),
            espec(0), espec(1), espec(2), espec(3),
        ],
        out_specs=pl.BlockSpec((pblk, 2 * d), lambda i: (i, 0)),
        out_shape=jax.ShapeDtypeStruct((quarter, 2 * d), jnp.int32),
    )(base_token.reshape(d, 1), emb_t, emb_t, emb_t, emb_t)


def _remap_idx(actions_t, v, quarter):
    def body(a_ref, o_ref):
        a = a_ref[...]
        safe = jnp.where(a >= 0, a, jnp.int32(v))
        m = (safe % quarter) * 4 + safe // quarter
        o_ref[...] = pltpu.einshape("s(kp)->s(pk)", m, k=4)

    seq, bsz = actions_t.shape
    sblk = 8
    return pl.pallas_call(
        body,
        grid=(seq // sblk,),
        in_specs=[pl.BlockSpec((sblk, bsz), lambda i: (i, 0))],
        out_specs=pl.BlockSpec((sblk, bsz), lambda i: (i, 0)),
        out_shape=jax.ShapeDtypeStruct(actions_t.shape, jnp.int32),
    )(actions_t)


def _sc_gather(table, idx2d, n, half):
    n_w = n // _NW
    group = _CH * _GR
    n_groups = n_w // group

    mesh = plsc.VectorSubcoreMesh(core_axis_name="c", subcore_axis_name="s")

    @functools.partial(
        pl.kernel,
        out_type=jax.ShapeDtypeStruct((n, half), jnp.int32),
        mesh=mesh,
        scratch_types=[
            pltpu.VMEM((2, _GR, _CH), jnp.int32),
            pltpu.VMEM((2, group, half), jnp.int32),
            pltpu.SemaphoreType.DMA,
            pltpu.SemaphoreType.DMA,
            pltpu.SemaphoreType.DMA,
            pltpu.SemaphoreType.DMA,
        ],
        compiler_params=pltpu.CompilerParams(use_tc_tiling_on_sc=False),
    )
    def k(table_hbm, idx_hbm, out_hbm, idx_v, rows_v, sg0, sg1, sw0, sw1):
        wid = lax.axis_index("s") * _NC + lax.axis_index("c")
        base = wid * n_w
        sg = (sg0, sg1)
        sw = (sw0, sw1)

        def gbase_of(g):
            return pl.multiple_of(base + g * group, group)

        def stage_fire(g, p):
            gbase = gbase_of(g)
            pltpu.sync_copy(
                idx_hbm.at[pl.ds(pl.multiple_of(gbase // _CH, _GR), _GR)],
                idx_v.at[p],
            )
            for j in range(_GR):
                pltpu.async_copy(
                    table_hbm.at[idx_v.at[p].at[j]],
                    rows_v.at[p].at[pl.ds(j * _CH, _CH)],
                    sg[p],
                )

        def drain_gathers(p):
            for j in range(_GR):
                pltpu.make_async_copy(
                    table_hbm.at[idx_v.at[p].at[j]],
                    rows_v.at[p].at[pl.ds(j * _CH, _CH)],
                    sg[p],
                ).wait()

        def wb_async(g, p):
            pltpu.async_copy(rows_v.at[p], out_hbm.at[pl.ds(gbase_of(g), group)], sw[p])

        def wb_wait(g, p):
            pltpu.make_async_copy(
                rows_v.at[p], out_hbm.at[pl.ds(gbase_of(g), group)], sw[p]
            ).wait()

        stage_fire(0, 0)

        def pair(i, carry):
            for j in (0, 1):
                g = 2 * i + 1 + j
                p = (1 + j) % 2
                q = 1 - p
                if j == 0:
                    @pl.when(i > 0)
                    def _():
                        wb_wait(g - 2, p)
                else:
                    wb_wait(g - 2, p)
                stage_fire(g, p)
                drain_gathers(q)
                wb_async(g - 1, q)
            return carry

        lax.fori_loop(0, (n_groups - 1) // 2, pair, 0)

        last = n_groups - 1
        drain_gathers(last % 2)
        wb_wait(last - 1, 1 - last % 2)
        pltpu.sync_copy(
            rows_v.at[last % 2], out_hbm.at[pl.ds(gbase_of(last), group)]
        )

    return k(table, idx2d)


def _unpack(packed128, bsz, seq, d, sblk):
    half = d // 2
    rps = bsz * half // 128
    qg = 128 // half

    bq = bsz // qg

    def body(x_ref, o_ref):
        for s in range(sblk):
            xt = x_ref[pl.ds(s * rps, rps), :].T
            for k in range(qg):
                w = xt[k * half:(k + 1) * half, :]
                o_ref[s, :, pl.ds(k * bq, bq)] = pltpu.bitcast(w, jnp.bfloat16)

    return pl.pallas_call(
        body,
        grid=(seq // sblk,),
        in_specs=[pl.BlockSpec((sblk * rps, 128), lambda i: (i, 0))],
        out_specs=pl.BlockSpec((sblk, d, bsz), lambda i: (i, 0, 0)),
        out_shape=jax.ShapeDtypeStruct((seq, d, bsz), jnp.bfloat16),
    )(packed128)


def kernel(actions, batch_time_shape, base_token, embedding):
    d = base_token.shape[0]
    if actions is None:
        batch_size, seq_len = batch_time_shape
        bt = base_token.astype(jnp.bfloat16)
        return jnp.broadcast_to(bt, (batch_size, seq_len, d))

    batch_size, seq_len = actions.shape
    v = embedding.shape[0]
    n = batch_size * seq_len
    half = d // 2

    v_pad = 100352
    quarter = v_pad // 4

    table128 = _prep_table(base_token, embedding.T, v, v_pad, pblk=3584)
    table = table128.reshape(v_pad, half)

    idx_perm = _remap_idx(jnp.asarray(actions, jnp.int32).T, v, quarter)
    idx2d = idx_perm.reshape(n // _CH, _CH)

    packed = _sc_gather(table, idx2d, n, half)
    packed128 = packed.reshape(n * half // 128, 128)
    out_t = _unpack(packed128, batch_size, seq_len, d, sblk=8)
    return jnp.transpose(out_t, (2, 0, 1))


# speedup vs baseline: 10.8737x; 1.0027x over previous
import functools

import jax
import jax.numpy as jnp
from jax import lax
from jax.experimental import pallas as pl
from jax.experimental.pallas import tpu as pltpu
from jax.experimental.pallas import tpu_sc as plsc

_NC = 2
_NS = 16
_NW = _NC * _NS

_CH = 128
_GR = 8


def _prep_table(base_token, emb_t, v, v_pad, pblk):
    d = emb_t.shape[0]
    quarter = v_pad // 4
    nblk = quarter // pblk

    def body(bt_ref, e0, e1, e2, e3, out_ref):
        i = pl.program_id(0)
        bt_bf = bt_ref[...].astype(jnp.bfloat16)
        for k, e_ref in enumerate((e0, e1, e2, e3)):
            rows = (
                jax.lax.broadcasted_iota(jnp.int32, (1, pblk), 1)
                + (k * quarter + i * pblk)
            )
            e_bf = e_ref[...].astype(jnp.bfloat16)
            z = jnp.where(rows < v, e_bf, jnp.zeros((), jnp.bfloat16)) + bt_bf
            w = pltpu.bitcast(z, jnp.int32)
            out_ref[:, k * (d // 2):(k + 1) * (d // 2)] = w.T

    def espec(k):
        return pl.BlockSpec((d, pblk), lambda i, k=k: (0, k * nblk + i))

    return pl.pallas_call(
        body,
        grid=(nblk,),
        in_specs=[
            pl.BlockSpec((d, 1), lambda i: (0, 0)),
            espec(0), espec(1), espec(2), espec(3),
        ],
        out_specs=pl.BlockSpec((pblk, 2 * d), lambda i: (i, 0)),
        out_shape=jax.ShapeDtypeStruct((quarter, 2 * d), jnp.int32),
    )(base_token.reshape(d, 1), emb_t, emb_t, emb_t, emb_t)


def _remap_idx(actions_t, v, quarter):
    def body(a_ref, o_ref):
        a = a_ref[...]
        safe = jnp.where(a >= 0, a, jnp.int32(v))
        m = (safe % quarter) * 4 + safe // quarter
        o_ref[...] = pltpu.einshape("s(kp)->s(pk)", m, k=4)

    seq, bsz = actions_t.shape
    sblk = 8
    return pl.pallas_call(
        body,
        grid=(seq // sblk,),
        in_specs=[pl.BlockSpec((sblk, bsz), lambda i: (i, 0))],
        out_specs=pl.BlockSpec((sblk, bsz), lambda i: (i, 0)),
        out_shape=jax.ShapeDtypeStruct(actions_t.shape, jnp.int32),
    )(actions_t)


def _sc_gather(table, idx2d, n, half):
    n_w = n // _NW
    group = _CH * _GR
    n_groups = n_w // group

    mesh = plsc.VectorSubcoreMesh(core_axis_name="c", subcore_axis_name="s")

    @functools.partial(
        pl.kernel,
        out_type=jax.ShapeDtypeStruct((n, half), jnp.int32),
        mesh=mesh,
        scratch_types=[
            pltpu.VMEM((2, _GR, _CH), jnp.int32),
            pltpu.VMEM((2, group, half), jnp.int32),
            pltpu.SemaphoreType.DMA,
            pltpu.SemaphoreType.DMA,
            pltpu.SemaphoreType.DMA,
            pltpu.SemaphoreType.DMA,
        ],
        compiler_params=pltpu.CompilerParams(use_tc_tiling_on_sc=False),
    )
    def k(table_hbm, idx_hbm, out_hbm, idx_v, rows_v, sg0, sg1, sw0, sw1):
        wid = lax.axis_index("s") * _NC + lax.axis_index("c")
        base = wid * n_w
        sg = (sg0, sg1)
        sw = (sw0, sw1)

        def gbase_of(g):
            return pl.multiple_of(base + g * group, group)

        def stage_fire(g, p):
            gbase = gbase_of(g)
            pltpu.sync_copy(
                idx_hbm.at[pl.ds(pl.multiple_of(gbase // _CH, _GR), _GR)],
                idx_v.at[p],
            )
            for j in range(_GR):
                pltpu.async_copy(
                    table_hbm.at[idx_v.at[p].at[j]],
                    rows_v.at[p].at[pl.ds(j * _CH, _CH)],
                    sg[p],
                )

        def drain_gathers(p):
            for j in range(_GR):
                pltpu.make_async_copy(
                    table_hbm.at[idx_v.at[p].at[j]],
                    rows_v.at[p].at[pl.ds(j * _CH, _CH)],
                    sg[p],
                ).wait()

        def wb_async(g, p):
            pltpu.async_copy(rows_v.at[p], out_hbm.at[pl.ds(gbase_of(g), group)], sw[p])

        def wb_wait(g, p):
            pltpu.make_async_copy(
                rows_v.at[p], out_hbm.at[pl.ds(gbase_of(g), group)], sw[p]
            ).wait()

        stage_fire(0, 0)

        def pair(i, carry):
            for j in (0, 1):
                g = 2 * i + 1 + j
                p = (1 + j) % 2
                q = 1 - p
                if j == 0:
                    @pl.when(i > 0)
                    def _():
                        wb_wait(g - 2, p)
                else:
                    wb_wait(g - 2, p)
                stage_fire(g, p)
                drain_gathers(q)
                wb_async(g - 1, q)
            return carry

        lax.fori_loop(0, (n_groups - 1) // 2, pair, 0)

        last = n_groups - 1
        drain_gathers(last % 2)
        wb_wait(last - 1, 1 - last % 2)
        pltpu.sync_copy(
            rows_v.at[last % 2], out_hbm.at[pl.ds(gbase_of(last), group)]
        )

    return k(table, idx2d)


def _unpack(packed128, bsz, seq, d, sblk):
    half = d // 2
    rps = bsz * half // 128
    qg = 128 // half

    bq = bsz // qg

    def body(x_ref, o_ref):
        for s in range(sblk):
            xt = x_ref[pl.ds(s * rps, rps), :].T
            for k in range(qg):
                w = xt[k * half:(k + 1) * half, :]
                o_ref[s, :, pl.ds(k * bq, bq)] = pltpu.bitcast(w, jnp.bfloat16)

    return pl.pallas_call(
        body,
        grid=(seq // sblk,),
        in_specs=[pl.BlockSpec((sblk * rps, 128), lambda i: (i, 0))],
        out_specs=pl.BlockSpec((sblk, d, bsz), lambda i: (i, 0, 0)),
        out_shape=jax.ShapeDtypeStruct((seq, d, bsz), jnp.bfloat16),
    )(packed128)


def kernel(actions, batch_time_shape, base_token, embedding):
    d = base_token.shape[0]
    if actions is None:
        batch_size, seq_len = batch_time_shape
        bt = base_token.astype(jnp.bfloat16)
        return jnp.broadcast_to(bt, (batch_size, seq_len, d))

    batch_size, seq_len = actions.shape
    v = embedding.shape[0]
    n = batch_size * seq_len
    half = d // 2

    v_pad = 100352
    quarter = v_pad // 4

    table128 = _prep_table(base_token, embedding.T, v, v_pad, pblk=3584)
    table = table128.reshape(v_pad, half)

    idx_perm = _remap_idx(jnp.asarray(actions, jnp.int32).T, v, quarter)
    idx2d = idx_perm.reshape(n // _CH, _CH)

    packed = _sc_gather(table, idx2d, n, half)
    packed128 = packed.reshape(n * half // 128, 128)
    out_t = _unpack(packed128, batch_size, seq_len, d, sblk=10)
    return jnp.transpose(out_t, (2, 0, 1))
